# baseline (speedup 1.0000x reference)
import functools

import jax
import jax.numpy as jnp
from jax import lax
from jax.experimental import pallas as pl
from jax.experimental.pallas import tpu as pltpu
from jax.experimental.pallas import tpu_sc as plsc

VOCAB_ROWS = 1000000
EMBED_DIM = 64
BATCH = 4096
FIELDS = 26
B = BATCH * FIELDS
NC, NS = 2, 16
NW = NC * NS
B_PER_W = B // NW
N_CHUNKS = 8
CH = B_PER_W // N_CHUNKS

T_BLK = 1024
T_GRID = (VOCAB_ROWS + T_BLK - 1) // T_BLK


def _tc_transpose(table_t):
    def body(in_ref, out_ref):
        out_ref[:, 0:EMBED_DIM] = in_ref[...].T

    return pl.pallas_call(
        body,
        grid=(T_GRID,),
        in_specs=[pl.BlockSpec((EMBED_DIM, T_BLK), lambda g: (0, g))],
        out_specs=pl.BlockSpec((T_BLK, 128), lambda g: (g, 0)),
        out_shape=jax.ShapeDtypeStruct((VOCAB_ROWS, 128), jnp.float32),
    )(table_t)


def _sc_gather(idx_flat, lin):
    mesh = plsc.VectorSubcoreMesh(core_axis_name="c", subcore_axis_name="s")

    @functools.partial(
        pl.kernel,
        mesh=mesh,
        out_type=jax.ShapeDtypeStruct((B, 128), jnp.float32),
        scratch_types=[
            pltpu.VMEM((B_PER_W,), jnp.int32),
            pltpu.VMEM((CH, 128), jnp.float32),
            pltpu.VMEM((CH, 128), jnp.float32),
            pltpu.SemaphoreType.DMA,
            pltpu.SemaphoreType.DMA,
        ],
    )
    def k(idx_hbm, lin_hbm, out_hbm, idx_v, buf0, buf1, sem0, sem1):
        wid = lax.axis_index("s") * NC + lax.axis_index("c")
        base = wid * B_PER_W
        pltpu.sync_copy(idx_hbm.at[pl.ds(base, B_PER_W)], idx_v)
        bufs = (buf0, buf1)
        sems = (sem0, sem1)
        cps = [None] * N_CHUNKS
        cps[0] = pltpu.async_copy(
            lin_hbm.at[idx_v.at[pl.ds(0, CH)]], bufs[0], sems[0])
        for i in range(N_CHUNKS):
            if i + 1 < N_CHUNKS:
                cps[i + 1] = pltpu.async_copy(
                    lin_hbm.at[idx_v.at[pl.ds((i + 1) * CH, CH)]],
                    bufs[(i + 1) % 2], sems[(i + 1) % 2])
            cps[i].wait()
            pltpu.sync_copy(bufs[i % 2], out_hbm.at[pl.ds(base + i * CH, CH)])

    return k(idx_flat, lin)


def kernel(inputs, embedding):
    table_t = embedding.T
    lin = _tc_transpose(table_t)
    idx = inputs.reshape(-1).astype(jnp.int32)
    out = _sc_gather(idx, lin)
    return out[:, :EMBED_DIM].reshape(BATCH, FIELDS, EMBED_DIM)


# speedup vs baseline: 2.0902x; 2.0902x over previous
import functools

import jax
import jax.numpy as jnp
from jax import lax
from jax.experimental import pallas as pl
from jax.experimental.pallas import tpu as pltpu
from jax.experimental.pallas import tpu_sc as plsc

VOCAB_ROWS = 1000000
EMBED_DIM = 64
BATCH = 4096
FIELDS = 26
B = BATCH * FIELDS
NC, NS = 2, 16
NW = NC * NS
B_PER_W = B // NW
N_CHUNKS = 8
CH = B_PER_W // N_CHUNKS

T_BLK = 8192
T_GRID = (VOCAB_ROWS + T_BLK - 1) // T_BLK


def _tc_transpose(table_t):
    def body(in_ref, out_ref):
        out_ref[:, 0:EMBED_DIM] = in_ref[...].T

    return pl.pallas_call(
        body,
        grid=(T_GRID,),
        in_specs=[pl.BlockSpec((EMBED_DIM, T_BLK), lambda g: (0, g))],
        out_specs=pl.BlockSpec((T_BLK, 128), lambda g: (g, 0)),
        out_shape=jax.ShapeDtypeStruct((VOCAB_ROWS, 128), jnp.float32),
    )(table_t)


def _sc_gather(idx_flat, lin):
    mesh = plsc.VectorSubcoreMesh(core_axis_name="c", subcore_axis_name="s")

    @functools.partial(
        pl.kernel,
        mesh=mesh,
        out_type=jax.ShapeDtypeStruct((B, 128), jnp.float32),
        scratch_types=[
            pltpu.VMEM((B_PER_W,), jnp.int32),
            pltpu.VMEM((CH, 128), jnp.float32),
            pltpu.VMEM((CH, 128), jnp.float32),
            pltpu.SemaphoreType.DMA,
            pltpu.SemaphoreType.DMA,
        ],
    )
    def k(idx_hbm, lin_hbm, out_hbm, idx_v, buf0, buf1, sem0, sem1):
        wid = lax.axis_index("s") * NC + lax.axis_index("c")
        base = wid * B_PER_W
        pltpu.sync_copy(idx_hbm.at[pl.ds(base, B_PER_W)], idx_v)
        bufs = (buf0, buf1)
        sems = (sem0, sem1)
        cps = [None] * N_CHUNKS
        cps[0] = pltpu.async_copy(
            lin_hbm.at[idx_v.at[pl.ds(0, CH)]], bufs[0], sems[0])
        for i in range(N_CHUNKS):
            if i + 1 < N_CHUNKS:
                cps[i + 1] = pltpu.async_copy(
                    lin_hbm.at[idx_v.at[pl.ds((i + 1) * CH, CH)]],
                    bufs[(i + 1) % 2], sems[(i + 1) % 2])
            cps[i].wait()
            pltpu.sync_copy(bufs[i % 2], out_hbm.at[pl.ds(base + i * CH, CH)])

    return k(idx_flat, lin)


def kernel(inputs, embedding):
    table_t = embedding.T
    lin = _tc_transpose(table_t)
    idx = inputs.reshape(-1).astype(jnp.int32)
    out = _sc_gather(idx, lin)
    return out[:, :EMBED_DIM].reshape(BATCH, FIELDS, EMBED_DIM)
